# Initial kernel scaffold; baseline (speedup 1.0000x reference)
import jax
import jax.numpy as jnp
from jax.experimental import pallas as pl


def kernel(hidden_states, router_logits):
    raise NotImplementedError("write your pallas kernel here")


# speedup vs baseline: 1.1081x; 1.1081x over previous
import functools

import jax
import jax.numpy as jnp
from jax.experimental import pallas as pl

NUM_TOKENS = 32768
NUM_EXPERTS = 64
TOP_K = 8
BLOCK = 1024


def _router_kernel(logits_ref, w_ref, id_ref):
    x = logits_ref[...]
    b = x.shape[0]
    col8 = jax.lax.broadcasted_iota(jnp.int32, (b, TOP_K), 1)
    lane = jax.lax.broadcasted_iota(jnp.int32, (b, NUM_EXPERTS), 1)
    vals = jnp.zeros((b, TOP_K), dtype=jnp.float32)
    ids = jnp.zeros((b, TOP_K), dtype=jnp.int32)
    cur = x
    for j in range(TOP_K):
        m = jnp.max(cur, axis=-1, keepdims=True)
        a = jnp.argmax(cur, axis=-1).astype(jnp.int32)
        a2 = a[:, None]
        vals = jnp.where(col8 == j, m, vals)
        ids = jnp.where(col8 == j, a2, ids)
        cur = jnp.where(lane == a2, -jnp.inf, cur)
    e = jnp.exp(vals - vals[:, 0:1])
    w_ref[...] = e / jnp.sum(e, axis=-1, keepdims=True)
    id_ref[...] = ids


@functools.partial(jax.jit, static_argnames=())
def _route(router_logits):
    grid = (NUM_TOKENS // BLOCK,)
    return pl.pallas_call(
        _router_kernel,
        grid=grid,
        in_specs=[pl.BlockSpec((BLOCK, NUM_EXPERTS), lambda i: (i, 0))],
        out_specs=[
            pl.BlockSpec((BLOCK, TOP_K), lambda i: (i, 0)),
            pl.BlockSpec((BLOCK, TOP_K), lambda i: (i, 0)),
        ],
        out_shape=[
            jax.ShapeDtypeStruct((NUM_TOKENS, TOP_K), jnp.float32),
            jax.ShapeDtypeStruct((NUM_TOKENS, TOP_K), jnp.int32),
        ],
    )(router_logits)


def kernel(hidden_states, router_logits):
    topk_weights, topk_ids = _route(router_logits)
    return hidden_states, topk_weights, topk_ids


# speedup vs baseline: 1.4378x; 1.2975x over previous
import jax
import jax.numpy as jnp
from jax.experimental import pallas as pl

NUM_TOKENS = 32768
HIDDEN = 2048
NUM_EXPERTS = 64
TOP_K = 8
BLOCK = 1024


def _fused_kernel(h_ref, logits_ref, h_out_ref, w_ref, id_ref):
    h_out_ref[...] = h_ref[...]
    x = logits_ref[...]
    b = x.shape[0]
    col8 = jax.lax.broadcasted_iota(jnp.int32, (b, TOP_K), 1)
    lane = jax.lax.broadcasted_iota(jnp.int32, (b, NUM_EXPERTS), 1)
    vals = jnp.zeros((b, TOP_K), dtype=jnp.float32)
    ids = jnp.zeros((b, TOP_K), dtype=jnp.int32)
    cur = x
    for j in range(TOP_K):
        m = jnp.max(cur, axis=-1, keepdims=True)
        a = jnp.argmax(cur, axis=-1).astype(jnp.int32)
        a2 = a[:, None]
        vals = jnp.where(col8 == j, m, vals)
        ids = jnp.where(col8 == j, a2, ids)
        cur = jnp.where(lane == a2, -jnp.inf, cur)
    e = jnp.exp(vals - vals[:, 0:1])
    w_ref[...] = e / jnp.sum(e, axis=-1, keepdims=True)
    id_ref[...] = ids


def kernel(hidden_states, router_logits):
    grid = (NUM_TOKENS // BLOCK,)
    h_out, topk_weights, topk_ids = pl.pallas_call(
        _fused_kernel,
        grid=grid,
        in_specs=[
            pl.BlockSpec((BLOCK, HIDDEN), lambda i: (i, 0)),
            pl.BlockSpec((BLOCK, NUM_EXPERTS), lambda i: (i, 0)),
        ],
        out_specs=[
            pl.BlockSpec((BLOCK, HIDDEN), lambda i: (i, 0)),
            pl.BlockSpec((BLOCK, TOP_K), lambda i: (i, 0)),
            pl.BlockSpec((BLOCK, TOP_K), lambda i: (i, 0)),
        ],
        out_shape=[
            jax.ShapeDtypeStruct((NUM_TOKENS, HIDDEN), jnp.float32),
            jax.ShapeDtypeStruct((NUM_TOKENS, TOP_K), jnp.float32),
            jax.ShapeDtypeStruct((NUM_TOKENS, TOP_K), jnp.int32),
        ],
    )(hidden_states, router_logits)
    return h_out, topk_weights, topk_ids
